# baseline (speedup 1.0000x reference)
import functools

import jax
import jax.numpy as jnp
from jax import lax
from jax.experimental import pallas as pl
from jax.experimental.pallas import tpu as pltpu
from jax.experimental.pallas import tpu_sc as plsc

N_NODES = 10000
N_EDGES = 160000
D_FEAT = 256
HALF = 128
ALPHA1 = 0.005
ALPHA2 = 1.0
BETA1 = 1.0
BETA2 = 0.1
EPS = 1e-15

NT = 16
CHUNK = 128
NCHUNK = 80
E_PER_TILE = CHUNK * NCHUNK
E_PAD = NT * E_PER_TILE
NBUF = 2
ROW_BYTES = CHUNK * HALF * 4
IDX_BYTES = CHUNK * 4
GB = 5
RB = 2048
N_PAD_ROWS = GB * RB
ROWS_PER_TILE = 624

@functools.cache
def _make_sc_edge_aggregate():
    mesh = plsc.VectorSubcoreMesh(core_axis_name="c", subcore_axis_name="s")

    scratch = [
        pltpu.VMEM_SHARED((N_NODES, HALF), jnp.float32),
        pltpu.VMEM((NCHUNK, CHUNK), jnp.int32),
        pltpu.SemaphoreType.DMA,
    ]
    for _ in range(NBUF):
        scratch += [
            pltpu.VMEM((CHUNK,), jnp.int32),
            pltpu.VMEM((CHUNK,), jnp.float32),
            pltpu.VMEM((CHUNK, HALF), jnp.float32),
            pltpu.SemaphoreType.DMA,
            pltpu.SemaphoreType.DMA,
            pltpu.SemaphoreType.DMA,
        ]

    @functools.partial(
        pl.kernel,
        mesh=mesh,
        out_type=(
            jax.ShapeDtypeStruct((N_PAD_ROWS, HALF), jnp.float32),
            jax.ShapeDtypeStruct((N_PAD_ROWS, HALF), jnp.float32),
        ),
        scratch_types=scratch,
    )
    def _sc_edge_aggregate(feat2, src_h2, dst_h, em, out_lo, out_hi,
                           acc, src_all, sem_h, *bufs):
        _sc_body(feat2, src_h2, dst_h, em, out_lo, out_hi,
                 acc, src_all, sem_h, bufs)

    return _sc_edge_aggregate


def _sc_body(feat2, src_h2, dst_h, em, out_lo, out_hi,
             acc, src_all, sem_h, bufs):
    c = lax.axis_index("c")
    s = lax.axis_index("s")
    B = [bufs[i * 6:(i + 1) * 6] for i in range(NBUF)]
    rows0 = B[0][2]

    def _zrow(i, carry):
        for j in range(HALF // 16):
            rows0[i, pl.ds(j * 16, 16)] = jnp.zeros((16,), jnp.float32)
        return carry
    lax.fori_loop(0, CHUNK, _zrow, 0)
    rbase = s * ROWS_PER_TILE
    _nf, _rem = divmod(ROWS_PER_TILE, CHUNK)
    for t in range(_nf):
        pltpu.sync_copy(rows0.at[pl.ds(0, CHUNK)],
                        acc.at[pl.ds(rbase + t * CHUNK, CHUNK)])
    if _rem:
        pltpu.sync_copy(rows0.at[pl.ds(0, _rem)],
                        acc.at[pl.ds(rbase + _nf * CHUNK, _rem)])

    @pl.when(s == NT - 1)
    def _():
        pltpu.sync_copy(rows0.at[pl.ds(0, N_NODES - NT * ROWS_PER_TILE)],
                        acc.at[pl.ds(NT * ROWS_PER_TILE,
                                     N_NODES - NT * ROWS_PER_TILE)])
    plsc.subcore_barrier()

    ebase = s * E_PER_TILE

    pltpu.async_copy(src_h2.at[pl.ds(s * NCHUNK, NCHUNK)], src_all,
                     sem_h).wait()

    def _prep(r, carry):
        for g in range(CHUNK // 16):
            sl = pl.ds(g * 16, 16)
            src_all[r, sl] = src_all[r, sl] * 2 + c
        return carry
    lax.fori_loop(0, NCHUNK, _prep, 0)

    def issue_de(ci, b):
        dst_r, em_r, _, sem_de, _, _ = B[b]
        base = ebase + ci * CHUNK
        pltpu.async_copy(dst_h.at[pl.ds(base, CHUNK)], dst_r, sem_de)
        pltpu.async_copy(em.at[pl.ds(base, CHUNK)], em_r, sem_de)

    def issue_gather(ci, b):
        _, _, rows_r, _, sem_g, _ = B[b]
        pltpu.async_copy(feat2.at[src_all.at[ci]], rows_r, sem_g)

    def issue_scatter(b):
        dst_r, _, rows_r, _, _, sem_s = B[b]
        pltpu.async_copy(rows_r, acc.at[dst_r], sem_s, add=True)

    def scale(b):
        _, em_r, rows_r, _, _, _ = B[b]

        def _scale(g, carry):
            x = em_r[pl.ds(g * 16, 16)]
            wv = 1.0 / (1.0 + jnp.exp(-x))
            for l in range(16):
                w = lax.gather(
                    wv, jnp.full((16, 1), l, jnp.int32),
                    lax.GatherDimensionNumbers(offset_dims=(),
                                               collapsed_slice_dims=(0,),
                                               start_index_map=(0,)),
                    (1,), mode=lax.GatherScatterMode.PROMISE_IN_BOUNDS)
                e = g * 16 + l
                for j in range(HALF // 16):
                    slj = pl.ds(j * 16, 16)
                    rows_r[e, slj] = rows_r[e, slj] * w
            return carry
        lax.fori_loop(0, CHUNK // 16, _scale, 0)

    issue_de(0, 0)
    issue_gather(0, 0)

    def _round(k, carry):
        for b in range(NBUF):
            ci = k * NBUF + b
            nxt = ci + 1
            nb = (b + 1) % NBUF
            dst_r, em_r, rows_r, sem_de, sem_g, sem_s = B[b]
            pltpu.make_async_copy(feat2.at[src_all.at[ci]], rows_r,
                                  sem_g).wait()

            @pl.when(nxt < NCHUNK)
            def _():
                @pl.when(nxt >= NBUF)
                def _():
                    pltpu.make_async_copy(
                        B[nb][2], acc.at[B[nb][0]], B[nb][5]).wait()
                issue_de(nxt, nb)
                issue_gather(nxt, nb)

            base = ebase + ci * CHUNK
            pltpu.make_async_copy(dst_h.at[pl.ds(base, CHUNK)], dst_r,
                                  sem_de).wait()
            pltpu.make_async_copy(em.at[pl.ds(base, CHUNK)], em_r,
                                  sem_de).wait()
            scale(b)
            issue_scatter(b)
        return carry
    lax.fori_loop(0, NCHUNK // NBUF, _round, 0)
    for b in range(NBUF):
        pltpu.make_async_copy(B[b][2], acc.at[B[b][0]], B[b][5]).wait()

    plsc.subcore_barrier()
    sl = pl.ds(rbase, ROWS_PER_TILE)
    sl_r = pl.ds(NT * ROWS_PER_TILE, N_NODES - NT * ROWS_PER_TILE)

    @pl.when(c == 0)
    def _():
        pltpu.sync_copy(acc.at[sl], out_lo.at[sl])

        @pl.when(s == NT - 1)
        def _():
            pltpu.sync_copy(acc.at[sl_r], out_lo.at[sl_r])

    @pl.when(c == 1)
    def _():
        pltpu.sync_copy(acc.at[sl], out_hi.at[sl])

        @pl.when(s == NT - 1)
        def _():
            pltpu.sync_copy(acc.at[sl_r], out_hi.at[sl_r])

    @pl.when(s == NT - 2)
    def _():
        def _zrow2(i, carry):
            for j in range(HALF // 16):
                rows0[i, pl.ds(j * 16, 16)] = jnp.zeros((16,), jnp.float32)
            return carry
        lax.fori_loop(0, CHUNK, _zrow2, 0)
        npad = N_PAD_ROWS - N_NODES
        nf, rem = divmod(npad, CHUNK)
        zspans = [(N_NODES + t * CHUNK, CHUNK) for t in range(nf)]
        if rem:
            zspans.append((N_NODES + nf * CHUNK, rem))

        @pl.when(c == 0)
        def _():
            for start, size in zspans:
                pltpu.sync_copy(rows0.at[pl.ds(0, size)],
                                out_lo.at[pl.ds(start, size)])

        @pl.when(c == 1)
        def _():
            for start, size in zspans:
                pltpu.sync_copy(rows0.at[pl.ds(0, size)],
                                out_hi.at[pl.ds(start, size)])


EB = N_EDGES // GB


def _tc_body(alo, ahi, em, pred, fm, w1, w2r, out):
    i = pl.program_id(0)

    @pl.when(i == 0)
    def _():
        out[0, 0] = 0.0

    sfm = 1.0 / (1.0 + jnp.exp(-fm[...]))
    hid = lax.dot_general(
        alo[...] * sfm[:, :HALF], w1[:HALF, :],
        (((1,), (0,)), ((), ())),
        precision=lax.Precision.HIGHEST, preferred_element_type=jnp.float32,
    ) + lax.dot_general(
        ahi[...] * sfm[:, HALF:], w1[HALF:, :],
        (((1,), (0,)), ((), ())),
        precision=lax.Precision.HIGHEST, preferred_element_type=jnp.float32,
    )
    hid = jnp.maximum(hid, 0.0)
    lg = lax.dot_general(
        hid, w2r[...], (((1,), (1,)), ((), ())),
        precision=lax.Precision.HIGHEST, preferred_element_type=jnp.float32,
    )[:, 0]
    mse_part = jnp.sum((lg - pred[...].reshape(RB)) ** 2) / N_NODES

    ew = 1.0 / (1.0 + jnp.exp(-em[...]))
    ent_e = -ew * jnp.log(ew + EPS) - (1.0 - ew) * jnp.log(1.0 - ew + EPS)
    contrib = mse_part + ALPHA1 * jnp.sum(ew) + ALPHA2 * jnp.sum(ent_e) / N_EDGES

    out[0, 0] += contrib

    @pl.when(i == GB - 1)
    def _():
        ent_f = -sfm * jnp.log(sfm + EPS) - (1.0 - sfm) * jnp.log(1.0 - sfm + EPS)
        out[0, 0] += BETA1 * jnp.mean(sfm) + BETA2 * jnp.mean(ent_f)


_tc_loss = pl.pallas_call(
    _tc_body,
    grid=(GB,),
    in_specs=[
        pl.BlockSpec((RB, HALF), lambda i: (i, 0)),
        pl.BlockSpec((RB, HALF), lambda i: (i, 0)),
        pl.BlockSpec((1, EB // 128, 128), lambda i: (i, 0, 0)),
        pl.BlockSpec((1, RB // 128, 128), lambda i: (i, 0, 0)),
        pl.BlockSpec((1, D_FEAT), lambda i: (0, 0)),
        pl.BlockSpec((D_FEAT, D_FEAT), lambda i: (0, 0)),
        pl.BlockSpec((1, D_FEAT), lambda i: (0, 0)),
    ],
    out_specs=pl.BlockSpec((1, 1), lambda i: (0, 0), memory_space=pltpu.SMEM),
    out_shape=jax.ShapeDtypeStruct((1, 1), jnp.float32),
)


def kernel(feat, edge_index, feat_mask, edge_mask, W1, W2, pred_value):
    feat2 = feat.reshape(2 * N_NODES, HALF)
    epad = E_PAD - N_EDGES
    src_h2 = jnp.concatenate(
        [edge_index[0], jnp.zeros((epad,), jnp.int32)]
    ).reshape(NT * NCHUNK, CHUNK)
    dst_h = jnp.concatenate([edge_index[1], jnp.zeros((epad,), jnp.int32)])
    em_h = jnp.concatenate(
        [edge_mask, jnp.full((epad,), -88.0, jnp.float32)])
    alo, ahi = _make_sc_edge_aggregate()(feat2, src_h2, dst_h, em_h)
    pad_n = N_PAD_ROWS - N_NODES
    pred_p = jnp.pad(pred_value, (0, pad_n)).reshape(GB, RB // 128, 128)
    em2 = edge_mask.reshape(GB, EB // 128, 128)
    w2r = W2.reshape(1, D_FEAT)
    out = _tc_loss(alo, ahi, em2, pred_p, feat_mask, W1, w2r)
    return out[0, 0]


# speedup vs baseline: 1.0004x; 1.0004x over previous
import functools

import jax
import jax.numpy as jnp
from jax import lax
from jax.experimental import pallas as pl
from jax.experimental.pallas import tpu as pltpu
from jax.experimental.pallas import tpu_sc as plsc

N_NODES = 10000
N_EDGES = 160000
D_FEAT = 256
HALF = 128
ALPHA1 = 0.005
ALPHA2 = 1.0
BETA1 = 1.0
BETA2 = 0.1
EPS = 1e-15

NT = 16
CHUNK = 128
NCHUNK = 80
E_PER_TILE = CHUNK * NCHUNK
E_PAD = NT * E_PER_TILE
NBUF = 2
ROW_BYTES = CHUNK * HALF * 4
IDX_BYTES = CHUNK * 4
GB = 10
RB = 1024
N_PAD_ROWS = GB * RB
ROWS_PER_TILE = 624

@functools.cache
def _make_sc_edge_aggregate():
    mesh = plsc.VectorSubcoreMesh(core_axis_name="c", subcore_axis_name="s")

    scratch = [
        pltpu.VMEM_SHARED((N_NODES, HALF), jnp.float32),
        pltpu.VMEM((NCHUNK, CHUNK), jnp.int32),
        pltpu.SemaphoreType.DMA,
    ]
    for _ in range(NBUF):
        scratch += [
            pltpu.VMEM((CHUNK,), jnp.int32),
            pltpu.VMEM((CHUNK,), jnp.float32),
            pltpu.VMEM((CHUNK, HALF), jnp.float32),
            pltpu.SemaphoreType.DMA,
            pltpu.SemaphoreType.DMA,
            pltpu.SemaphoreType.DMA,
        ]

    @functools.partial(
        pl.kernel,
        mesh=mesh,
        out_type=(
            jax.ShapeDtypeStruct((N_PAD_ROWS, HALF), jnp.float32),
            jax.ShapeDtypeStruct((N_PAD_ROWS, HALF), jnp.float32),
        ),
        scratch_types=scratch,
    )
    def _sc_edge_aggregate(feat2, src_h2, dst_h, em, out_lo, out_hi,
                           acc, src_all, sem_h, *bufs):
        _sc_body(feat2, src_h2, dst_h, em, out_lo, out_hi,
                 acc, src_all, sem_h, bufs)

    return _sc_edge_aggregate


def _sc_body(feat2, src_h2, dst_h, em, out_lo, out_hi,
             acc, src_all, sem_h, bufs):
    c = lax.axis_index("c")
    s = lax.axis_index("s")
    B = [bufs[i * 6:(i + 1) * 6] for i in range(NBUF)]
    rows0 = B[0][2]

    def _zrow(i, carry):
        for j in range(HALF // 16):
            rows0[i, pl.ds(j * 16, 16)] = jnp.zeros((16,), jnp.float32)
        return carry
    lax.fori_loop(0, CHUNK, _zrow, 0)
    rbase = s * ROWS_PER_TILE
    _nf, _rem = divmod(ROWS_PER_TILE, CHUNK)
    for t in range(_nf):
        pltpu.sync_copy(rows0.at[pl.ds(0, CHUNK)],
                        acc.at[pl.ds(rbase + t * CHUNK, CHUNK)])
    if _rem:
        pltpu.sync_copy(rows0.at[pl.ds(0, _rem)],
                        acc.at[pl.ds(rbase + _nf * CHUNK, _rem)])

    @pl.when(s == NT - 1)
    def _():
        pltpu.sync_copy(rows0.at[pl.ds(0, N_NODES - NT * ROWS_PER_TILE)],
                        acc.at[pl.ds(NT * ROWS_PER_TILE,
                                     N_NODES - NT * ROWS_PER_TILE)])
    plsc.subcore_barrier()

    ebase = s * E_PER_TILE

    pltpu.async_copy(src_h2.at[pl.ds(s * NCHUNK, NCHUNK)], src_all,
                     sem_h).wait()

    def _prep(r, carry):
        for g in range(CHUNK // 16):
            sl = pl.ds(g * 16, 16)
            src_all[r, sl] = src_all[r, sl] * 2 + c
        return carry
    lax.fori_loop(0, NCHUNK, _prep, 0)

    def issue_de(ci, b):
        dst_r, em_r, _, sem_de, _, _ = B[b]
        base = ebase + ci * CHUNK
        pltpu.async_copy(dst_h.at[pl.ds(base, CHUNK)], dst_r, sem_de)
        pltpu.async_copy(em.at[pl.ds(base, CHUNK)], em_r, sem_de)

    def issue_gather(ci, b):
        _, _, rows_r, _, sem_g, _ = B[b]
        pltpu.async_copy(feat2.at[src_all.at[ci]], rows_r, sem_g)

    def issue_scatter(b):
        dst_r, _, rows_r, _, _, sem_s = B[b]
        pltpu.async_copy(rows_r, acc.at[dst_r], sem_s, add=True)

    def scale(b):
        _, em_r, rows_r, _, _, _ = B[b]

        def _scale(g, carry):
            x = em_r[pl.ds(g * 16, 16)]
            wv = 1.0 / (1.0 + jnp.exp(-x))
            for l in range(16):
                w = lax.gather(
                    wv, jnp.full((16, 1), l, jnp.int32),
                    lax.GatherDimensionNumbers(offset_dims=(),
                                               collapsed_slice_dims=(0,),
                                               start_index_map=(0,)),
                    (1,), mode=lax.GatherScatterMode.PROMISE_IN_BOUNDS)
                e = g * 16 + l
                for j in range(HALF // 16):
                    slj = pl.ds(j * 16, 16)
                    rows_r[e, slj] = rows_r[e, slj] * w
            return carry
        lax.fori_loop(0, CHUNK // 16, _scale, 0)

    issue_de(0, 0)
    issue_gather(0, 0)

    def _round(k, carry):
        for b in range(NBUF):
            ci = k * NBUF + b
            nxt = ci + 1
            nb = (b + 1) % NBUF
            dst_r, em_r, rows_r, sem_de, sem_g, sem_s = B[b]
            pltpu.make_async_copy(feat2.at[src_all.at[ci]], rows_r,
                                  sem_g).wait()

            @pl.when(nxt < NCHUNK)
            def _():
                @pl.when(nxt >= NBUF)
                def _():
                    pltpu.make_async_copy(
                        B[nb][2], acc.at[B[nb][0]], B[nb][5]).wait()
                issue_de(nxt, nb)
                issue_gather(nxt, nb)

            base = ebase + ci * CHUNK
            pltpu.make_async_copy(dst_h.at[pl.ds(base, CHUNK)], dst_r,
                                  sem_de).wait()
            pltpu.make_async_copy(em.at[pl.ds(base, CHUNK)], em_r,
                                  sem_de).wait()
            scale(b)
            issue_scatter(b)
        return carry
    lax.fori_loop(0, NCHUNK // NBUF, _round, 0)
    for b in range(NBUF):
        pltpu.make_async_copy(B[b][2], acc.at[B[b][0]], B[b][5]).wait()

    plsc.subcore_barrier()
    sl = pl.ds(rbase, ROWS_PER_TILE)
    sl_r = pl.ds(NT * ROWS_PER_TILE, N_NODES - NT * ROWS_PER_TILE)

    @pl.when(c == 0)
    def _():
        pltpu.sync_copy(acc.at[sl], out_lo.at[sl])

        @pl.when(s == NT - 1)
        def _():
            pltpu.sync_copy(acc.at[sl_r], out_lo.at[sl_r])

    @pl.when(c == 1)
    def _():
        pltpu.sync_copy(acc.at[sl], out_hi.at[sl])

        @pl.when(s == NT - 1)
        def _():
            pltpu.sync_copy(acc.at[sl_r], out_hi.at[sl_r])

    @pl.when(s == NT - 2)
    def _():
        def _zrow2(i, carry):
            for j in range(HALF // 16):
                rows0[i, pl.ds(j * 16, 16)] = jnp.zeros((16,), jnp.float32)
            return carry
        lax.fori_loop(0, CHUNK, _zrow2, 0)
        npad = N_PAD_ROWS - N_NODES
        nf, rem = divmod(npad, CHUNK)
        zspans = [(N_NODES + t * CHUNK, CHUNK) for t in range(nf)]
        if rem:
            zspans.append((N_NODES + nf * CHUNK, rem))

        @pl.when(c == 0)
        def _():
            for start, size in zspans:
                pltpu.sync_copy(rows0.at[pl.ds(0, size)],
                                out_lo.at[pl.ds(start, size)])

        @pl.when(c == 1)
        def _():
            for start, size in zspans:
                pltpu.sync_copy(rows0.at[pl.ds(0, size)],
                                out_hi.at[pl.ds(start, size)])


EB = N_EDGES // GB


def _tc_body(alo, ahi, em, pred, fm, w1, w2r, out):
    i = pl.program_id(0)

    @pl.when(i == 0)
    def _():
        out[0, 0] = 0.0

    sfm = 1.0 / (1.0 + jnp.exp(-fm[...]))
    hid = lax.dot_general(
        alo[...] * sfm[:, :HALF], w1[:HALF, :],
        (((1,), (0,)), ((), ())),
        precision=lax.Precision.HIGHEST, preferred_element_type=jnp.float32,
    ) + lax.dot_general(
        ahi[...] * sfm[:, HALF:], w1[HALF:, :],
        (((1,), (0,)), ((), ())),
        precision=lax.Precision.HIGHEST, preferred_element_type=jnp.float32,
    )
    hid = jnp.maximum(hid, 0.0)
    lg = lax.dot_general(
        hid, w2r[...], (((1,), (1,)), ((), ())),
        precision=lax.Precision.HIGHEST, preferred_element_type=jnp.float32,
    )[:, 0]
    mse_part = jnp.sum((lg - pred[...].reshape(RB)) ** 2) / N_NODES

    ew = 1.0 / (1.0 + jnp.exp(-em[...]))
    ent_e = -ew * jnp.log(ew + EPS) - (1.0 - ew) * jnp.log(1.0 - ew + EPS)
    contrib = mse_part + ALPHA1 * jnp.sum(ew) + ALPHA2 * jnp.sum(ent_e) / N_EDGES

    out[0, 0] += contrib

    @pl.when(i == GB - 1)
    def _():
        ent_f = -sfm * jnp.log(sfm + EPS) - (1.0 - sfm) * jnp.log(1.0 - sfm + EPS)
        out[0, 0] += BETA1 * jnp.mean(sfm) + BETA2 * jnp.mean(ent_f)


_tc_loss = pl.pallas_call(
    _tc_body,
    grid=(GB,),
    in_specs=[
        pl.BlockSpec((RB, HALF), lambda i: (i, 0)),
        pl.BlockSpec((RB, HALF), lambda i: (i, 0)),
        pl.BlockSpec((1, EB // 128, 128), lambda i: (i, 0, 0)),
        pl.BlockSpec((1, RB // 128, 128), lambda i: (i, 0, 0)),
        pl.BlockSpec((1, D_FEAT), lambda i: (0, 0)),
        pl.BlockSpec((D_FEAT, D_FEAT), lambda i: (0, 0)),
        pl.BlockSpec((1, D_FEAT), lambda i: (0, 0)),
    ],
    out_specs=pl.BlockSpec((1, 1), lambda i: (0, 0), memory_space=pltpu.SMEM),
    out_shape=jax.ShapeDtypeStruct((1, 1), jnp.float32),
)


def kernel(feat, edge_index, feat_mask, edge_mask, W1, W2, pred_value):
    feat2 = feat.reshape(2 * N_NODES, HALF)
    epad = E_PAD - N_EDGES
    src_h2 = jnp.concatenate(
        [edge_index[0], jnp.zeros((epad,), jnp.int32)]
    ).reshape(NT * NCHUNK, CHUNK)
    dst_h = jnp.concatenate([edge_index[1], jnp.zeros((epad,), jnp.int32)])
    em_h = jnp.concatenate(
        [edge_mask, jnp.full((epad,), -88.0, jnp.float32)])
    alo, ahi = _make_sc_edge_aggregate()(feat2, src_h2, dst_h, em_h)
    pad_n = N_PAD_ROWS - N_NODES
    pred_p = jnp.pad(pred_value, (0, pad_n)).reshape(GB, RB // 128, 128)
    em2 = edge_mask.reshape(GB, EB // 128, 128)
    w2r = W2.reshape(1, D_FEAT)
    out = _tc_loss(alo, ahi, em2, pred_p, feat_mask, W1, w2r)
    return out[0, 0]


# speedup vs baseline: 1.1599x; 1.1594x over previous
import functools

import jax
import jax.numpy as jnp
from jax import lax
from jax.experimental import pallas as pl
from jax.experimental.pallas import tpu as pltpu
from jax.experimental.pallas import tpu_sc as plsc

N_NODES = 10000
N_EDGES = 160000
D_FEAT = 256
HALF = 128
ALPHA1 = 0.005
ALPHA2 = 1.0
BETA1 = 1.0
BETA2 = 0.1
EPS = 1e-15

NT = 16
CHUNK = 128
NCHUNK = 80
E_PER_TILE = CHUNK * NCHUNK
E_PAD = NT * E_PER_TILE
NBUF = 2
ROW_BYTES = CHUNK * HALF * 4
IDX_BYTES = CHUNK * 4
GB = 10
RB = 1024
N_PAD_ROWS = GB * RB
ROWS_PER_TILE = 624

@functools.cache
def _make_sc_edge_aggregate():
    mesh = plsc.VectorSubcoreMesh(core_axis_name="c", subcore_axis_name="s")

    scratch = [
        pltpu.VMEM_SHARED((N_NODES, HALF), jnp.float32),
        pltpu.VMEM((NCHUNK, CHUNK), jnp.int32),
        pltpu.SemaphoreType.DMA,
    ]
    for _ in range(NBUF):
        scratch += [
            pltpu.VMEM((CHUNK,), jnp.int32),
            pltpu.VMEM((CHUNK,), jnp.float32),
            pltpu.VMEM((CHUNK, HALF), jnp.float32),
            pltpu.SemaphoreType.DMA,
            pltpu.SemaphoreType.DMA,
            pltpu.SemaphoreType.DMA,
        ]

    @functools.partial(
        pl.kernel,
        mesh=mesh,
        out_type=(
            jax.ShapeDtypeStruct((N_NODES, HALF), jnp.float32),
            jax.ShapeDtypeStruct((N_NODES, HALF), jnp.float32),
        ),
        scratch_types=scratch,
    )
    def _sc_edge_aggregate(feat2, src_h2, dst_h, em, out_lo, out_hi,
                           acc, src_all, sem_h, *bufs):
        _sc_body(feat2, src_h2, dst_h, em, out_lo, out_hi,
                 acc, src_all, sem_h, bufs)

    return _sc_edge_aggregate


def _sc_body(feat2, src_h2, dst_h, em, out_lo, out_hi,
             acc, src_all, sem_h, bufs):
    c = lax.axis_index("c")
    s = lax.axis_index("s")
    B = [bufs[i * 6:(i + 1) * 6] for i in range(NBUF)]
    rows0 = B[0][2]

    def _zrow(i, carry):
        for j in range(HALF // 16):
            rows0[i, pl.ds(j * 16, 16)] = jnp.zeros((16,), jnp.float32)
        return carry
    lax.fori_loop(0, CHUNK, _zrow, 0)
    rbase = s * ROWS_PER_TILE
    _nf, _rem = divmod(ROWS_PER_TILE, CHUNK)
    for t in range(_nf):
        pltpu.sync_copy(rows0.at[pl.ds(0, CHUNK)],
                        acc.at[pl.ds(rbase + t * CHUNK, CHUNK)])
    if _rem:
        pltpu.sync_copy(rows0.at[pl.ds(0, _rem)],
                        acc.at[pl.ds(rbase + _nf * CHUNK, _rem)])

    @pl.when(s == NT - 1)
    def _():
        pltpu.sync_copy(rows0.at[pl.ds(0, N_NODES - NT * ROWS_PER_TILE)],
                        acc.at[pl.ds(NT * ROWS_PER_TILE,
                                     N_NODES - NT * ROWS_PER_TILE)])
    plsc.subcore_barrier()

    ebase = s * E_PER_TILE

    pltpu.async_copy(src_h2.at[pl.ds(s * NCHUNK, NCHUNK)], src_all,
                     sem_h).wait()

    def _prep(r, carry):
        for g in range(CHUNK // 16):
            sl = pl.ds(g * 16, 16)
            src_all[r, sl] = src_all[r, sl] * 2 + c
        return carry
    lax.fori_loop(0, NCHUNK, _prep, 0)

    def issue_de(ci, b):
        dst_r, em_r, _, sem_de, _, _ = B[b]
        base = ebase + ci * CHUNK
        pltpu.async_copy(dst_h.at[pl.ds(base, CHUNK)], dst_r, sem_de)
        pltpu.async_copy(em.at[pl.ds(base, CHUNK)], em_r, sem_de)

    def issue_gather(ci, b):
        _, _, rows_r, _, sem_g, _ = B[b]
        pltpu.async_copy(feat2.at[src_all.at[ci]], rows_r, sem_g)

    def issue_scatter(b):
        dst_r, _, rows_r, _, _, sem_s = B[b]
        pltpu.async_copy(rows_r, acc.at[dst_r], sem_s, add=True)

    def scale(b):
        _, em_r, rows_r, _, _, _ = B[b]

        def _scale(g, carry):
            x = em_r[pl.ds(g * 16, 16)]
            wv = 1.0 / (1.0 + jnp.exp(-x))
            for l in range(16):
                w = lax.gather(
                    wv, jnp.full((16, 1), l, jnp.int32),
                    lax.GatherDimensionNumbers(offset_dims=(),
                                               collapsed_slice_dims=(0,),
                                               start_index_map=(0,)),
                    (1,), mode=lax.GatherScatterMode.PROMISE_IN_BOUNDS)
                e = g * 16 + l
                for j in range(HALF // 16):
                    slj = pl.ds(j * 16, 16)
                    rows_r[e, slj] = rows_r[e, slj] * w
            return carry
        lax.fori_loop(0, CHUNK // 16, _scale, 0)

    issue_de(0, 0)
    issue_gather(0, 0)

    def _round(k, carry):
        for b in range(NBUF):
            ci = k * NBUF + b
            nxt = ci + 1
            nb = (b + 1) % NBUF
            dst_r, em_r, rows_r, sem_de, sem_g, sem_s = B[b]
            pltpu.make_async_copy(feat2.at[src_all.at[ci]], rows_r,
                                  sem_g).wait()

            @pl.when(nxt < NCHUNK)
            def _():
                @pl.when(nxt >= NBUF)
                def _():
                    pltpu.make_async_copy(
                        B[nb][2], acc.at[B[nb][0]], B[nb][5]).wait()
                issue_de(nxt, nb)
                issue_gather(nxt, nb)

            base = ebase + ci * CHUNK
            pltpu.make_async_copy(dst_h.at[pl.ds(base, CHUNK)], dst_r,
                                  sem_de).wait()
            pltpu.make_async_copy(em.at[pl.ds(base, CHUNK)], em_r,
                                  sem_de).wait()
            scale(b)
            issue_scatter(b)
        return carry
    lax.fori_loop(0, NCHUNK // NBUF, _round, 0)
    for b in range(NBUF):
        pltpu.make_async_copy(B[b][2], acc.at[B[b][0]], B[b][5]).wait()

    plsc.subcore_barrier()
    sl = pl.ds(rbase, ROWS_PER_TILE)
    sl_r = pl.ds(NT * ROWS_PER_TILE, N_NODES - NT * ROWS_PER_TILE)

    @pl.when(c == 0)
    def _():
        pltpu.sync_copy(acc.at[sl], out_lo.at[sl])

        @pl.when(s == NT - 1)
        def _():
            pltpu.sync_copy(acc.at[sl_r], out_lo.at[sl_r])

    @pl.when(c == 1)
    def _():
        pltpu.sync_copy(acc.at[sl], out_hi.at[sl])

        @pl.when(s == NT - 1)
        def _():
            pltpu.sync_copy(acc.at[sl_r], out_hi.at[sl_r])

EB = N_EDGES // GB


def _tc_body(alo, ahi, em, pred, fm, w1, w2r, out):
    i = pl.program_id(0)

    @pl.when(i == 0)
    def _():
        out[0, 0] = 0.0

    sfm = 1.0 / (1.0 + jnp.exp(-fm[...]))
    hid = lax.dot_general(
        alo[...] * sfm[:, :HALF], w1[:HALF, :],
        (((1,), (0,)), ((), ())),
        precision=lax.Precision.HIGHEST, preferred_element_type=jnp.float32,
    ) + lax.dot_general(
        ahi[...] * sfm[:, HALF:], w1[HALF:, :],
        (((1,), (0,)), ((), ())),
        precision=lax.Precision.HIGHEST, preferred_element_type=jnp.float32,
    )
    hid = jnp.maximum(hid, 0.0)
    lg = lax.dot_general(
        hid, w2r[...], (((1,), (1,)), ((), ())),
        precision=lax.Precision.HIGHEST, preferred_element_type=jnp.float32,
    )[:, 0]
    mse_part = jnp.sum((lg - pred[...].reshape(RB)) ** 2) / N_NODES

    ew = 1.0 / (1.0 + jnp.exp(-em[...]))
    ent_e = -ew * jnp.log(ew + EPS) - (1.0 - ew) * jnp.log(1.0 - ew + EPS)
    contrib = mse_part + ALPHA1 * jnp.sum(ew) + ALPHA2 * jnp.sum(ent_e) / N_EDGES

    out[0, 0] += contrib

    @pl.when(i == GB - 1)
    def _():
        ent_f = -sfm * jnp.log(sfm + EPS) - (1.0 - sfm) * jnp.log(1.0 - sfm + EPS)
        out[0, 0] += BETA1 * jnp.mean(sfm) + BETA2 * jnp.mean(ent_f)


_tc_loss = pl.pallas_call(
    _tc_body,
    grid=(GB,),
    in_specs=[
        pl.BlockSpec((RB, HALF), lambda i: (i, 0)),
        pl.BlockSpec((RB, HALF), lambda i: (i, 0)),
        pl.BlockSpec((1, EB // 128, 128), lambda i: (i, 0, 0)),
        pl.BlockSpec((1, RB // 128, 128), lambda i: (i, 0, 0)),
        pl.BlockSpec((1, D_FEAT), lambda i: (0, 0)),
        pl.BlockSpec((D_FEAT, D_FEAT), lambda i: (0, 0)),
        pl.BlockSpec((1, D_FEAT), lambda i: (0, 0)),
    ],
    out_specs=pl.BlockSpec((1, 1), lambda i: (0, 0), memory_space=pltpu.SMEM),
    out_shape=jax.ShapeDtypeStruct((1, 1), jnp.float32),
)


def kernel(feat, edge_index, feat_mask, edge_mask, W1, W2, pred_value):
    feat2 = feat.reshape(2 * N_NODES, HALF)
    epad = E_PAD - N_EDGES
    src_h2 = jnp.concatenate(
        [edge_index[0], jnp.zeros((epad,), jnp.int32)]
    ).reshape(NT * NCHUNK, CHUNK)
    dst_h = jnp.concatenate([edge_index[1], jnp.zeros((epad,), jnp.int32)])
    em_h = jnp.concatenate(
        [edge_mask, jnp.full((epad,), -88.0, jnp.float32)])
    alo, ahi = _make_sc_edge_aggregate()(feat2, src_h2, dst_h, em_h)
    pad_n = N_PAD_ROWS - N_NODES
    alo_p = jnp.pad(alo, ((0, pad_n), (0, 0)))
    ahi_p = jnp.pad(ahi, ((0, pad_n), (0, 0)))
    pred_p = jnp.pad(pred_value, (0, pad_n)).reshape(GB, RB // 128, 128)
    em2 = edge_mask.reshape(GB, EB // 128, 128)
    w2r = W2.reshape(1, D_FEAT)
    out = _tc_loss(alo_p, ahi_p, em2, pred_p, feat_mask, W1, w2r)
    return out[0, 0]


# speedup vs baseline: 1.2174x; 1.0496x over previous
import functools

import jax
import jax.numpy as jnp
from jax import lax
from jax.experimental import pallas as pl
from jax.experimental.pallas import tpu as pltpu
from jax.experimental.pallas import tpu_sc as plsc

N_NODES = 10000
N_EDGES = 160000
D_FEAT = 256
HALF = 128
ALPHA1 = 0.005
ALPHA2 = 1.0
BETA1 = 1.0
BETA2 = 0.1
EPS = 1e-15

NT = 16
CHUNK = 128
NCHUNK = 80
E_PER_TILE = CHUNK * NCHUNK
E_PAD = NT * E_PER_TILE
NBUF = 2
ROW_BYTES = CHUNK * HALF * 4
IDX_BYTES = CHUNK * 4
GB = 10
RB = 1024
N_PAD_ROWS = GB * RB
ROWS_PER_TILE = 624

@functools.cache
def _make_sc_edge_aggregate():
    mesh = plsc.VectorSubcoreMesh(core_axis_name="c", subcore_axis_name="s")

    scratch = [
        pltpu.VMEM_SHARED((N_NODES, HALF), jnp.float32),
        pltpu.VMEM((NCHUNK, CHUNK), jnp.int32),
        pltpu.SemaphoreType.DMA,
    ]
    for _ in range(NBUF):
        scratch += [
            pltpu.VMEM((CHUNK,), jnp.int32),
            pltpu.VMEM((CHUNK,), jnp.float32),
            pltpu.VMEM((CHUNK, HALF), jnp.float32),
            pltpu.SemaphoreType.DMA,
            pltpu.SemaphoreType.DMA,
            pltpu.SemaphoreType.DMA,
        ]

    @functools.partial(
        pl.kernel,
        mesh=mesh,
        out_type=(
            jax.ShapeDtypeStruct((N_NODES, HALF), jnp.float32),
            jax.ShapeDtypeStruct((N_NODES, HALF), jnp.float32),
        ),
        scratch_types=scratch,
    )
    def _sc_edge_aggregate(feat2, src_h2, dst_h, em, out_lo, out_hi,
                           acc, src_all, sem_h, *bufs):
        _sc_body(feat2, src_h2, dst_h, em, out_lo, out_hi,
                 acc, src_all, sem_h, bufs)

    return _sc_edge_aggregate


def _sc_body(feat2, src_h2, dst_h, em, out_lo, out_hi,
             acc, src_all, sem_h, bufs):
    c = lax.axis_index("c")
    s = lax.axis_index("s")
    B = [bufs[i * 6:(i + 1) * 6] for i in range(NBUF)]
    rows0 = B[0][2]

    def _zrow(i, carry):
        for j in range(HALF // 16):
            rows0[i, pl.ds(j * 16, 16)] = jnp.zeros((16,), jnp.float32)
        return carry
    lax.fori_loop(0, CHUNK, _zrow, 0)
    rbase = s * ROWS_PER_TILE
    _nf, _rem = divmod(ROWS_PER_TILE, CHUNK)
    for t in range(_nf):
        pltpu.sync_copy(rows0.at[pl.ds(0, CHUNK)],
                        acc.at[pl.ds(rbase + t * CHUNK, CHUNK)])
    if _rem:
        pltpu.sync_copy(rows0.at[pl.ds(0, _rem)],
                        acc.at[pl.ds(rbase + _nf * CHUNK, _rem)])

    @pl.when(s == NT - 1)
    def _():
        pltpu.sync_copy(rows0.at[pl.ds(0, N_NODES - NT * ROWS_PER_TILE)],
                        acc.at[pl.ds(NT * ROWS_PER_TILE,
                                     N_NODES - NT * ROWS_PER_TILE)])
    plsc.subcore_barrier()

    ebase = s * E_PER_TILE

    pltpu.async_copy(src_h2.at[pl.ds(s * NCHUNK, NCHUNK)], src_all,
                     sem_h).wait()

    def _prep(r, carry):
        for g in range(CHUNK // 16):
            sl = pl.ds(g * 16, 16)
            src_all[r, sl] = src_all[r, sl] * 2 + c
        return carry
    lax.fori_loop(0, NCHUNK, _prep, 0)

    def issue_de(ci, b):
        dst_r, em_r, _, sem_de, _, _ = B[b]
        base = ebase + ci * CHUNK
        pltpu.async_copy(dst_h.at[pl.ds(base, CHUNK)], dst_r, sem_de)
        pltpu.async_copy(em.at[pl.ds(base, CHUNK)], em_r, sem_de)

    def issue_gather(ci, b):
        _, _, rows_r, _, sem_g, _ = B[b]
        pltpu.async_copy(feat2.at[src_all.at[ci]], rows_r, sem_g)

    def issue_scatter(b):
        dst_r, _, rows_r, _, _, sem_s = B[b]
        pltpu.async_copy(rows_r, acc.at[dst_r], sem_s, add=True)

    def scale(b):
        _, em_r, rows_r, _, _, _ = B[b]

        def _scale(g, carry):
            x = em_r[pl.ds(g * 16, 16)]
            wv = 1.0 / (1.0 + jnp.exp(-x))
            for l in range(16):
                w = lax.gather(
                    wv, jnp.full((16, 1), l, jnp.int32),
                    lax.GatherDimensionNumbers(offset_dims=(),
                                               collapsed_slice_dims=(0,),
                                               start_index_map=(0,)),
                    (1,), mode=lax.GatherScatterMode.PROMISE_IN_BOUNDS)
                e = g * 16 + l
                for j in range(HALF // 16):
                    slj = pl.ds(j * 16, 16)
                    rows_r[e, slj] = rows_r[e, slj] * w
            return carry
        lax.fori_loop(0, CHUNK // 16, _scale, 0)

    issue_de(0, 0)
    issue_gather(0, 0)

    def _round(k, carry):
        for b in range(NBUF):
            ci = k * NBUF + b
            nxt = ci + 1
            nb = (b + 1) % NBUF
            dst_r, em_r, rows_r, sem_de, sem_g, sem_s = B[b]
            pltpu.make_async_copy(feat2.at[src_all.at[ci]], rows_r,
                                  sem_g).wait()

            @pl.when(nxt < NCHUNK)
            def _():
                @pl.when(nxt >= NBUF)
                def _():
                    pltpu.make_async_copy(
                        B[nb][2], acc.at[B[nb][0]], B[nb][5]).wait()
                issue_de(nxt, nb)
                issue_gather(nxt, nb)

            base = ebase + ci * CHUNK
            pltpu.make_async_copy(dst_h.at[pl.ds(base, CHUNK)], dst_r,
                                  sem_de).wait()
            pltpu.make_async_copy(em.at[pl.ds(base, CHUNK)], em_r,
                                  sem_de).wait()
            scale(b)
            issue_scatter(b)
        return carry
    lax.fori_loop(0, NCHUNK // NBUF, _round, 0)
    for b in range(NBUF):
        pltpu.make_async_copy(B[b][2], acc.at[B[b][0]], B[b][5]).wait()

    plsc.subcore_barrier()
    sl = pl.ds(rbase, ROWS_PER_TILE)
    sl_r = pl.ds(NT * ROWS_PER_TILE, N_NODES - NT * ROWS_PER_TILE)

    @pl.when(c == 0)
    def _():
        pltpu.sync_copy(acc.at[sl], out_lo.at[sl])

        @pl.when(s == NT - 1)
        def _():
            pltpu.sync_copy(acc.at[sl_r], out_lo.at[sl_r])

    @pl.when(c == 1)
    def _():
        pltpu.sync_copy(acc.at[sl], out_hi.at[sl])

        @pl.when(s == NT - 1)
        def _():
            pltpu.sync_copy(acc.at[sl_r], out_hi.at[sl_r])

EB = N_EDGES // GB


def _tc_body(alo, ahi, em, pred, fm, w1, w2r, out):
    i = pl.program_id(0)

    @pl.when(i == 0)
    def _():
        out[0, 0] = 0.0

    sfm = 1.0 / (1.0 + jnp.exp(-fm[...]))
    hid = lax.dot_general(
        alo[...] * sfm[:, :HALF], w1[:HALF, :],
        (((1,), (0,)), ((), ())),
        preferred_element_type=jnp.float32,
    ) + lax.dot_general(
        ahi[...] * sfm[:, HALF:], w1[HALF:, :],
        (((1,), (0,)), ((), ())),
        preferred_element_type=jnp.float32,
    )
    hid = jnp.maximum(hid, 0.0)
    lg = lax.dot_general(
        hid, w2r[...], (((1,), (1,)), ((), ())),
        preferred_element_type=jnp.float32,
    )[:, 0]
    mse_part = jnp.sum((lg - pred[...].reshape(RB)) ** 2) / N_NODES

    ew = 1.0 / (1.0 + jnp.exp(-em[...]))
    ent_e = -ew * jnp.log(ew + EPS) - (1.0 - ew) * jnp.log(1.0 - ew + EPS)
    contrib = mse_part + ALPHA1 * jnp.sum(ew) + ALPHA2 * jnp.sum(ent_e) / N_EDGES

    out[0, 0] += contrib

    @pl.when(i == GB - 1)
    def _():
        ent_f = -sfm * jnp.log(sfm + EPS) - (1.0 - sfm) * jnp.log(1.0 - sfm + EPS)
        out[0, 0] += BETA1 * jnp.mean(sfm) + BETA2 * jnp.mean(ent_f)


_tc_loss = pl.pallas_call(
    _tc_body,
    grid=(GB,),
    in_specs=[
        pl.BlockSpec((RB, HALF), lambda i: (i, 0)),
        pl.BlockSpec((RB, HALF), lambda i: (i, 0)),
        pl.BlockSpec((1, EB // 128, 128), lambda i: (i, 0, 0)),
        pl.BlockSpec((1, RB // 128, 128), lambda i: (i, 0, 0)),
        pl.BlockSpec((1, D_FEAT), lambda i: (0, 0)),
        pl.BlockSpec((D_FEAT, D_FEAT), lambda i: (0, 0)),
        pl.BlockSpec((1, D_FEAT), lambda i: (0, 0)),
    ],
    out_specs=pl.BlockSpec((1, 1), lambda i: (0, 0), memory_space=pltpu.SMEM),
    out_shape=jax.ShapeDtypeStruct((1, 1), jnp.float32),
)


def kernel(feat, edge_index, feat_mask, edge_mask, W1, W2, pred_value):
    feat2 = feat.reshape(2 * N_NODES, HALF)
    epad = E_PAD - N_EDGES
    src_h2 = jnp.concatenate(
        [edge_index[0], jnp.zeros((epad,), jnp.int32)]
    ).reshape(NT * NCHUNK, CHUNK)
    dst_h = jnp.concatenate([edge_index[1], jnp.zeros((epad,), jnp.int32)])
    em_h = jnp.concatenate(
        [edge_mask, jnp.full((epad,), -88.0, jnp.float32)])
    alo, ahi = _make_sc_edge_aggregate()(feat2, src_h2, dst_h, em_h)
    pad_n = N_PAD_ROWS - N_NODES
    alo_p = jnp.pad(alo, ((0, pad_n), (0, 0)))
    ahi_p = jnp.pad(ahi, ((0, pad_n), (0, 0)))
    pred_p = jnp.pad(pred_value, (0, pad_n)).reshape(GB, RB // 128, 128)
    em2 = edge_mask.reshape(GB, EB // 128, 128)
    w2r = W2.reshape(1, D_FEAT)
    out = _tc_loss(alo_p, ahi_p, em2, pred_p, feat_mask, W1, w2r)
    return out[0, 0]
